# baseline (speedup 1.0000x reference)
import jax
import jax.numpy as jnp
from jax.experimental import pallas as pl
from jax.experimental.pallas import tpu as pltpu


def _pick_bm(n, target):
    bm = 8
    for c in range(8, min(n, target) + 1, 8):
        if n % c == 0:
            bm = c
    return bm


def _make_fused_kernel(num_m, bm):
    def _fused(a_ref, x_ref, w1_ref, b1_ref, w2_ref, b2_ref, w3_ref, b3_ref,
               w4_ref, b4_ref, o_ref, s0_ref, s1_ref):
        t = pl.program_id(0)

        @pl.when(t == 0)
        def _():
            s0_ref[...] = jnp.dot(x_ref[...], w1_ref[...],
                                  preferred_element_type=jnp.float32)

        @pl.when(t < num_m)
        def _():
            acc = jnp.dot(a_ref[0], s0_ref[...],
                          preferred_element_type=jnp.float32)
            h1 = jnp.maximum(acc + b1_ref[...], 0.0)
            s1_ref[pl.ds(t * bm, bm), :] = jnp.dot(
                h1, w2_ref[...], preferred_element_type=jnp.float32)

        @pl.when(t >= num_m)
        def _():
            acc = jnp.dot(a_ref[0], s1_ref[...],
                          preferred_element_type=jnp.float32)
            z = jax.nn.log_sigmoid(acc + b2_ref[...])
            d1 = jnp.maximum(
                jnp.dot(z, w3_ref[...], preferred_element_type=jnp.float32)
                + b3_ref[...], 0.0)
            o_ref[...] = (jnp.dot(d1, w4_ref[...],
                                  preferred_element_type=jnp.float32)
                          + b4_ref[...])

    return _fused


def kernel(x_all, adj, adj_drop, W_gc1, b_gc1, W_gc2, b_gc2,
           W_lin1, b_lin1, W_lin2, b_lin2):
    del adj
    n, nfeat = x_all.shape
    nhid = W_gc1.shape[1]

    b_gc1 = b_gc1.reshape(1, -1)
    b_gc2 = b_gc2.reshape(1, -1)
    b_lin1 = b_lin1.reshape(1, -1)
    b_lin2 = b_lin2.reshape(1, -1)

    bm = _pick_bm(n, 200)
    num_m = n // bm

    const = lambda t: (0, 0)
    out = pl.pallas_call(
        _make_fused_kernel(num_m, bm),
        grid=(2 * num_m,),
        in_specs=[
            pl.BlockSpec((1, bm, n), lambda t: (t // num_m, t % num_m, 0)),
            pl.BlockSpec((n, nfeat), const),
            pl.BlockSpec((nfeat, nhid), const),
            pl.BlockSpec((1, nhid), const),
            pl.BlockSpec((nhid, nhid), const),
            pl.BlockSpec((1, nhid), const),
            pl.BlockSpec((nhid, nhid), const),
            pl.BlockSpec((1, nhid), const),
            pl.BlockSpec((nhid, 1), const),
            pl.BlockSpec((1, 1), const),
        ],
        out_specs=pl.BlockSpec(
            (bm, 1), lambda t: (jnp.maximum(t - num_m, 0), 0)),
        out_shape=jax.ShapeDtypeStruct((n, 1), jnp.float32),
        scratch_shapes=[
            pltpu.VMEM((n, nhid), jnp.float32),
            pltpu.VMEM((n, nhid), jnp.float32),
        ],
        compiler_params=pltpu.CompilerParams(
            dimension_semantics=("arbitrary",)),
    )(adj_drop, x_all, W_gc1, b_gc1, W_gc2, b_gc2,
      W_lin1, b_lin1, W_lin2, b_lin2)

    return out.reshape(-1)


# speedup vs baseline: 1.0523x; 1.0523x over previous
import jax
import jax.numpy as jnp
from jax.experimental import pallas as pl
from jax.experimental.pallas import tpu as pltpu


def _pick_bm(n, target):
    bm = 8
    for c in range(8, min(n, target) + 1, 8):
        if n % c == 0:
            bm = c
    return bm


def _make_fused_kernel(num_m, bm):
    def _fused(a_ref, x_ref, w1_ref, b1_ref, w2_ref, b2_ref, w3_ref, b3_ref,
               w4_ref, b4_ref, o_ref, s0_ref, s1_ref):
        t = pl.program_id(0)

        @pl.when(t == 0)
        def _():
            s0_ref[...] = jnp.dot(x_ref[...], w1_ref[...],
                                  preferred_element_type=jnp.float32
                                  ).astype(jnp.bfloat16)

        @pl.when(t < num_m)
        def _():
            acc = jnp.dot(a_ref[0].astype(jnp.bfloat16), s0_ref[...],
                          preferred_element_type=jnp.float32)
            h1 = jnp.maximum(acc + b1_ref[...], 0.0)
            s1_ref[pl.ds(t * bm, bm), :] = jnp.dot(
                h1, w2_ref[...], preferred_element_type=jnp.float32
            ).astype(jnp.bfloat16)

        @pl.when(t >= num_m)
        def _():
            acc = jnp.dot(a_ref[0].astype(jnp.bfloat16), s1_ref[...],
                          preferred_element_type=jnp.float32)
            z = jax.nn.log_sigmoid(acc + b2_ref[...])
            d1 = jnp.maximum(
                jnp.dot(z, w3_ref[...], preferred_element_type=jnp.float32)
                + b3_ref[...], 0.0)
            o_ref[...] = (jnp.dot(d1, w4_ref[...],
                                  preferred_element_type=jnp.float32)
                          + b4_ref[...])

    return _fused


def kernel(x_all, adj, adj_drop, W_gc1, b_gc1, W_gc2, b_gc2,
           W_lin1, b_lin1, W_lin2, b_lin2):
    del adj
    n, nfeat = x_all.shape
    nhid = W_gc1.shape[1]

    b_gc1 = b_gc1.reshape(1, -1)
    b_gc2 = b_gc2.reshape(1, -1)
    b_lin1 = b_lin1.reshape(1, -1)
    b_lin2 = b_lin2.reshape(1, -1)

    bm = _pick_bm(n, 400)
    num_m = n // bm

    const = lambda t: (0, 0)
    out = pl.pallas_call(
        _make_fused_kernel(num_m, bm),
        grid=(2 * num_m,),
        in_specs=[
            pl.BlockSpec((1, bm, n), lambda t: (t // num_m, t % num_m, 0)),
            pl.BlockSpec((n, nfeat), const),
            pl.BlockSpec((nfeat, nhid), const),
            pl.BlockSpec((1, nhid), const),
            pl.BlockSpec((nhid, nhid), const),
            pl.BlockSpec((1, nhid), const),
            pl.BlockSpec((nhid, nhid), const),
            pl.BlockSpec((1, nhid), const),
            pl.BlockSpec((nhid, 1), const),
            pl.BlockSpec((1, 1), const),
        ],
        out_specs=pl.BlockSpec(
            (bm, 1), lambda t: (jnp.maximum(t - num_m, 0), 0)),
        out_shape=jax.ShapeDtypeStruct((n, 1), jnp.float32),
        scratch_shapes=[
            pltpu.VMEM((n, nhid), jnp.bfloat16),
            pltpu.VMEM((n, nhid), jnp.bfloat16),
        ],
        compiler_params=pltpu.CompilerParams(
            dimension_semantics=("arbitrary",)),
    )(adj_drop, x_all, W_gc1, b_gc1, W_gc2, b_gc2,
      W_lin1, b_lin1, W_lin2, b_lin2)

    return out.reshape(-1)


# speedup vs baseline: 1.0538x; 1.0014x over previous
import jax
import jax.numpy as jnp
from jax.experimental import pallas as pl
from jax.experimental.pallas import tpu as pltpu


def _pick_bm(n, target):
    bm = 8
    for c in range(8, min(n, target) + 1, 8):
        if n % c == 0:
            bm = c
    return bm


def _make_fused_kernel(num_m, bm):
    def _fused(a_ref, x_ref, w1_ref, b1_ref, w2_ref, b2_ref, w3_ref, b3_ref,
               w4_ref, b4_ref, o_ref, s0_ref, s1_ref):
        t = pl.program_id(0)

        @pl.when(t == 0)
        def _():
            s0_ref[...] = jnp.dot(x_ref[...], w1_ref[...],
                                  preferred_element_type=jnp.float32)

        @pl.when(t < num_m)
        def _():
            acc = jnp.dot(a_ref[0], s0_ref[...],
                          preferred_element_type=jnp.float32)
            h1 = jnp.maximum(acc + b1_ref[...], 0.0)
            s1_ref[pl.ds(t * bm, bm), :] = jnp.dot(
                h1, w2_ref[...], preferred_element_type=jnp.float32)

        @pl.when(t >= num_m)
        def _():
            acc = jnp.dot(a_ref[0], s1_ref[...],
                          preferred_element_type=jnp.float32)
            z = jax.nn.log_sigmoid(acc + b2_ref[...])
            d1 = jnp.maximum(
                jnp.dot(z, w3_ref[...], preferred_element_type=jnp.float32)
                + b3_ref[...], 0.0)
            o_ref[...] = (jnp.dot(d1, w4_ref[...],
                                  preferred_element_type=jnp.float32)
                          + b4_ref[...])

    return _fused


def kernel(x_all, adj, adj_drop, W_gc1, b_gc1, W_gc2, b_gc2,
           W_lin1, b_lin1, W_lin2, b_lin2):
    del adj
    n, nfeat = x_all.shape
    nhid = W_gc1.shape[1]

    b_gc1 = b_gc1.reshape(1, -1)
    b_gc2 = b_gc2.reshape(1, -1)
    b_lin1 = b_lin1.reshape(1, -1)
    b_lin2 = b_lin2.reshape(1, -1)

    bm = _pick_bm(n, 400)
    num_m = n // bm

    const = lambda t: (0, 0)
    out = pl.pallas_call(
        _make_fused_kernel(num_m, bm),
        grid=(2 * num_m,),
        in_specs=[
            pl.BlockSpec((1, bm, n), lambda t: (t // num_m, t % num_m, 0)),
            pl.BlockSpec((n, nfeat), const),
            pl.BlockSpec((nfeat, nhid), const),
            pl.BlockSpec((1, nhid), const),
            pl.BlockSpec((nhid, nhid), const),
            pl.BlockSpec((1, nhid), const),
            pl.BlockSpec((nhid, nhid), const),
            pl.BlockSpec((1, nhid), const),
            pl.BlockSpec((nhid, 1), const),
            pl.BlockSpec((1, 1), const),
        ],
        out_specs=pl.BlockSpec(
            (bm, 1), lambda t: (jnp.maximum(t - num_m, 0), 0)),
        out_shape=jax.ShapeDtypeStruct((n, 1), jnp.float32),
        scratch_shapes=[
            pltpu.VMEM((n, nhid), jnp.float32),
            pltpu.VMEM((n, nhid), jnp.float32),
        ],
        compiler_params=pltpu.CompilerParams(
            dimension_semantics=("arbitrary",)),
    )(adj_drop, x_all, W_gc1, b_gc1, W_gc2, b_gc2,
      W_lin1, b_lin1, W_lin2, b_lin2)

    return out.reshape(-1)
